# Initial kernel scaffold; baseline (speedup 1.0000x reference)
import jax
import jax.numpy as jnp
from jax.experimental import pallas as pl


def kernel(rwse, edge_index, edge_attr, W_src, W_dst, bn_gamma, bn_beta):
    raise NotImplementedError("write your pallas kernel here")


# speedup vs baseline: 5.6842x; 5.6842x over previous
import jax
import jax.numpy as jnp
from jax.experimental import pallas as pl
from jax.experimental.pallas import tpu as pltpu

B_G = 128
NPG = 64
N_NODES = B_G * NPG
EMB = 16
OUT = 64
ROWS = B_G * NPG * NPG


def _base_kernel(rwse_ref, ws_ref, wd_ref, out_ref, stats_ref):
    g = pl.program_id(0)
    a = jnp.dot(rwse_ref[...], ws_ref[...].T, preferred_element_type=jnp.float32)
    b = jnp.dot(rwse_ref[...], wd_ref[...].T, preferred_element_type=jnp.float32)
    base = a[:, None, :] + b[None, :, :]
    out_ref[...] = base.reshape(NPG * NPG, OUT)
    sa = jnp.sum(a, axis=0)
    sb = jnp.sum(b, axis=0)
    s1 = NPG * (sa + sb)
    s2 = NPG * jnp.sum(a * a, axis=0) + NPG * jnp.sum(b * b, axis=0) + 2.0 * sa * sb
    upd = jnp.concatenate(
        [s1[None, :], s2[None, :], jnp.zeros((6, OUT), jnp.float32)], axis=0)

    @pl.when(g == 0)
    def _():
        stats_ref[...] = jnp.zeros_like(stats_ref)

    stats_ref[...] += upd


def _norm_kernel(x_ref, sc_ref, bi_ref, o_ref):
    o_ref[...] = x_ref[...] * sc_ref[0:1, :] + bi_ref[0:1, :]


def kernel(rwse, edge_index, edge_attr, W_src, W_dst, bn_gamma, bn_beta):
    rwse = rwse.astype(jnp.float32)
    out_raw, stats = pl.pallas_call(
        _base_kernel,
        grid=(B_G,),
        in_specs=[
            pl.BlockSpec((NPG, EMB), lambda g: (g, 0)),
            pl.BlockSpec((OUT, EMB), lambda g: (0, 0)),
            pl.BlockSpec((OUT, EMB), lambda g: (0, 0)),
        ],
        out_specs=[
            pl.BlockSpec((NPG * NPG, OUT), lambda g: (g, 0)),
            pl.BlockSpec((8, OUT), lambda g: (0, 0)),
        ],
        out_shape=[
            jax.ShapeDtypeStruct((ROWS, OUT), jnp.float32),
            jax.ShapeDtypeStruct((8, OUT), jnp.float32),
        ],
        compiler_params=pltpu.CompilerParams(
            dimension_semantics=("arbitrary",)),
    )(rwse, W_src, W_dst)

    src = edge_index[0].astype(jnp.int32)
    dst = edge_index[1].astype(jnp.int32)
    pos = src * NPG + jnp.remainder(dst, NPG)
    out_scat = out_raw.at[pos].add(edge_attr)

    s1 = stats[0] + jnp.sum(edge_attr, axis=0)
    mean = s1 / ROWS
    ex2 = jnp.sum(out_scat * out_scat, axis=0) / ROWS
    var = ex2 - mean * mean
    scale = bn_gamma / jnp.sqrt(var + 1e-5)
    bias = bn_beta - mean * scale

    scale8 = jnp.broadcast_to(scale[None, :], (8, OUT))
    bias8 = jnp.broadcast_to(bias[None, :], (8, OUT))
    out_val = pl.pallas_call(
        _norm_kernel,
        grid=(B_G,),
        in_specs=[
            pl.BlockSpec((NPG * NPG, OUT), lambda g: (g, 0)),
            pl.BlockSpec((8, OUT), lambda g: (0, 0)),
            pl.BlockSpec((8, OUT), lambda g: (0, 0)),
        ],
        out_specs=pl.BlockSpec((NPG * NPG, OUT), lambda g: (g, 0)),
        out_shape=jax.ShapeDtypeStruct((ROWS, OUT), jnp.float32),
    )(out_scat, scale8, bias8)

    k = jnp.arange(ROWS, dtype=jnp.int32)
    n = k // NPG
    out_idx = jnp.stack([n, (n // NPG) * NPG + k % NPG])
    return out_idx, out_val
